# baseline (speedup 1.0000x reference)
import functools

import jax
import jax.numpy as jnp
from jax import lax
from jax.experimental import pallas as pl
from jax.experimental.pallas import tpu as pltpu
from jax.experimental.pallas import tpu_sc as plsc

D_MODEL = 64
SCALE = 8.0

NC = 2
NS = 16
NW = NC * NS

LANES = 16
NB = 4


def _make_sc_gather(n_rows: int, seq: int):
    rows_per_w = n_rows // NW
    n_chunks = rows_per_w
    n_blocks = n_chunks // NB
    split = [(0, 128), (128, seq - 128)] if seq > 128 else [(0, seq)]
    assert n_chunks % NB == 0 and n_blocks >= 2
    assert all(ln % 8 == 0 for _, ln in split)

    mesh = plsc.VectorSubcoreMesh(core_axis_name="c", subcore_axis_name="s")

    @functools.partial(
        pl.kernel,
        out_type=jax.ShapeDtypeStruct((n_rows, seq, D_MODEL), jnp.float32),
        mesh=mesh,
        scratch_types=[
            pltpu.VMEM((rows_per_w, seq), jnp.int32),
            pltpu.VMEM((NB, seq, D_MODEL), jnp.float32),
            pltpu.VMEM((NB, seq, D_MODEL), jnp.float32),
            [pltpu.SemaphoreType.DMA] * NB,
            [pltpu.SemaphoreType.DMA] * NB,
        ],
        compiler_params=pltpu.CompilerParams(use_tc_tiling_on_sc=False),
    )
    def sc_kernel(idx_hbm, table_hbm, out_hbm, idx_v, gbuf, sbuf, gsems, ssems):
        wid = lax.axis_index("s") * NC + lax.axis_index("c")
        row0 = wid * rows_per_w
        pltpu.sync_copy(idx_hbm.at[pl.ds(row0, rows_per_w)], idx_v)

        def fire_gather(chunk, b):
            for off, ln in split:
                pltpu.async_copy(
                    table_hbm.at[idx_v.at[chunk, pl.ds(off, ln)]],
                    gbuf.at[b, pl.ds(off, ln)], gsems[b])

        def wait_gather(b):
            for off, ln in split:
                pltpu.make_async_copy(
                    table_hbm.at[idx_v.at[0, pl.ds(0, ln)]],
                    gbuf.at[b, pl.ds(off, ln)], gsems[b]).wait()

        def fire_scatter(chunk, b):
            pltpu.async_copy(sbuf.at[b], out_hbm.at[row0 + chunk], ssems[b])

        def wait_scatter(b):
            pltpu.make_async_copy(sbuf.at[b], out_hbm.at[0], ssems[b]).wait()

        def scale(b):
            g, s = gbuf.at[b], sbuf.at[b]

            @plsc.parallel_loop(0, seq, unroll=4)
            def _(r):
                for q in range(D_MODEL // LANES):
                    sl = pl.ds(q * LANES, LANES)
                    s[r, sl] = g[r, sl] * SCALE

        def process(chunk, b, first, last):
            wait_gather(b)
            if not first:
                wait_scatter(b)
            scale(b)
            fire_scatter(chunk, b)
            if not last:
                fire_gather(chunk + NB, b)

        for b in range(NB):
            fire_gather(b, b)
        for b in range(NB):
            process(b, b, first=True, last=False)

        @pl.loop(1, n_blocks - 1)
        def _(j):
            base = j * NB
            for b in range(NB):
                process(base + b, b, first=False, last=False)

        for b in range(NB):
            process((n_blocks - 1) * NB + b, b, first=False, last=True)
        for b in range(NB):
            wait_scatter(b)

    return sc_kernel


def kernel(x, table):
    n_rows, seq = x.shape
    return _make_sc_gather(n_rows, seq)(x, table)


# speedup vs baseline: 1.1105x; 1.1105x over previous
import functools

import jax
import jax.numpy as jnp
from jax import lax
from jax.experimental import pallas as pl
from jax.experimental.pallas import tpu as pltpu
from jax.experimental.pallas import tpu_sc as plsc

D_MODEL = 64
PADDED = 128
SCALE = 8.0

NC = 2
NS = 16
NW = NC * NS

LANES = 16
NB = 2


def _make_sc_gather(n_rows: int, seq: int):
    rows_per_w = n_rows // NW
    n_chunks = rows_per_w
    n_blocks = n_chunks // NB
    split = [(0, 128), (128, seq - 128)] if seq > 128 else [(0, seq)]
    assert n_chunks % NB == 0 and n_blocks >= 2
    assert all(ln % 8 == 0 for _, ln in split)

    mesh = plsc.VectorSubcoreMesh(core_axis_name="c", subcore_axis_name="s")

    @functools.partial(
        pl.kernel,
        out_type=jax.ShapeDtypeStruct((n_rows, seq, D_MODEL), jnp.float32),
        mesh=mesh,
        scratch_types=[
            pltpu.VMEM((rows_per_w * seq,), jnp.int32),
            pltpu.VMEM((NB, seq, PADDED), jnp.float32),
            pltpu.VMEM((NB, seq, D_MODEL), jnp.float32),
            [pltpu.SemaphoreType.DMA] * NB,
            [pltpu.SemaphoreType.DMA] * NB,
        ],
        compiler_params=pltpu.CompilerParams(use_tc_tiling_on_sc=True),
    )
    def sc_kernel(idx_hbm, table_hbm, out_hbm, idx_v, gbuf, sbuf, gsems, ssems):
        wid = lax.axis_index("s") * NC + lax.axis_index("c")
        row0 = wid * rows_per_w
        pltpu.sync_copy(idx_hbm.at[pl.ds(row0 * seq, rows_per_w * seq)], idx_v)

        def fire_gather(chunk, b):
            for off, ln in split:
                pltpu.async_copy(
                    table_hbm.at[idx_v.at[pl.ds(chunk * seq + off, ln)]],
                    gbuf.at[b, pl.ds(off, ln)], gsems[b])

        def wait_gather(b):
            for off, ln in split:
                pltpu.make_async_copy(
                    table_hbm.at[idx_v.at[pl.ds(0, ln)]],
                    gbuf.at[b, pl.ds(off, ln)], gsems[b]).wait()

        def fire_scatter(chunk, b):
            pltpu.async_copy(sbuf.at[b], out_hbm.at[row0 + chunk], ssems[b])

        def wait_scatter(b):
            pltpu.make_async_copy(sbuf.at[b], out_hbm.at[0], ssems[b]).wait()

        def scale(b):
            g, s = gbuf.at[b], sbuf.at[b]

            @plsc.parallel_loop(0, seq, unroll=4)
            def _(r):
                for q in range(D_MODEL // LANES):
                    sl = pl.ds(q * LANES, LANES)
                    s[r, sl] = g[r, sl] * SCALE

        def process(chunk, b, first, last):
            wait_gather(b)
            if not first:
                wait_scatter(b)
            scale(b)
            fire_scatter(chunk, b)
            if not last:
                fire_gather(chunk + NB, b)

        for b in range(NB):
            fire_gather(b, b)
        for b in range(NB):
            process(b, b, first=True, last=False)

        @pl.loop(1, n_blocks - 1)
        def _(j):
            base = j * NB
            for b in range(NB):
                process(base + b, b, first=False, last=False)

        for b in range(NB):
            process((n_blocks - 1) * NB + b, b, first=False, last=True)
        for b in range(NB):
            wait_scatter(b)

    return sc_kernel


def kernel(x, table):
    n_rows, seq = x.shape
    table_p = jnp.pad(table, ((0, 0), (0, PADDED - D_MODEL)))
    return _make_sc_gather(n_rows, seq)(x.reshape(-1), table_p)


# speedup vs baseline: 1.2233x; 1.1016x over previous
import functools

import jax
import jax.numpy as jnp
from jax import lax
from jax.experimental import pallas as pl
from jax.experimental.pallas import tpu as pltpu
from jax.experimental.pallas import tpu_sc as plsc

D_MODEL = 64
PADDED = 128
SCALE = 8.0

NC = 2
NS = 16
NW = NC * NS

LANES = 16
NB = 2


def _make_sc_gather(n_rows: int, seq: int):
    rows_per_w = n_rows // NW
    n_chunks = rows_per_w
    n_blocks = n_chunks // NB
    split = [(0, 128), (128, seq - 128)] if seq > 128 else [(0, seq)]
    assert n_chunks % NB == 0 and n_blocks >= 2
    assert all(ln % 8 == 0 for _, ln in split)

    mesh = plsc.VectorSubcoreMesh(core_axis_name="c", subcore_axis_name="s")

    @functools.partial(
        pl.kernel,
        out_type=jax.ShapeDtypeStruct((n_rows * seq, D_MODEL), jnp.float32),
        mesh=mesh,
        scratch_types=[
            pltpu.VMEM((rows_per_w * seq,), jnp.int32),
            pltpu.VMEM((NB, seq, PADDED), jnp.float32),
            pltpu.VMEM((NB, seq, D_MODEL), jnp.float32),
            [pltpu.SemaphoreType.DMA] * NB,
            [pltpu.SemaphoreType.DMA] * NB,
        ],
        compiler_params=pltpu.CompilerParams(use_tc_tiling_on_sc=True),
    )
    def sc_kernel(idx_hbm, table_hbm, out_hbm, idx_v, gbuf, sbuf, gsems, ssems):
        wid = lax.axis_index("s") * NC + lax.axis_index("c")
        row0 = wid * rows_per_w
        pltpu.sync_copy(idx_hbm.at[pl.ds(row0 * seq, rows_per_w * seq)], idx_v)

        def fire_gather(chunk, b):
            for off, ln in split:
                pltpu.async_copy(
                    table_hbm.at[idx_v.at[pl.ds(chunk * seq + off, ln)]],
                    gbuf.at[b, pl.ds(off, ln)], gsems[b])

        def wait_gather(b):
            for off, ln in split:
                pltpu.make_async_copy(
                    table_hbm.at[idx_v.at[pl.ds(0, ln)]],
                    gbuf.at[b, pl.ds(off, ln)], gsems[b]).wait()

        def fire_scatter(chunk, b):
            pltpu.async_copy(
                sbuf.at[b], out_hbm.at[pl.ds((row0 + chunk) * seq, seq)],
                ssems[b])

        def wait_scatter(b):
            pltpu.make_async_copy(
                sbuf.at[b], out_hbm.at[pl.ds(0, seq)], ssems[b]).wait()

        def scale(b):
            g, s = gbuf.at[b], sbuf.at[b]

            @plsc.parallel_loop(0, seq, unroll=4)
            def _(r):
                for q in range(D_MODEL // LANES):
                    sl = pl.ds(q * LANES, LANES)
                    s[r, sl] = g[r, sl] * SCALE

        def process(chunk, b, first, last):
            wait_gather(b)
            if not first:
                wait_scatter(b)
            scale(b)
            fire_scatter(chunk, b)
            if not last:
                fire_gather(chunk + NB, b)

        for b in range(NB):
            fire_gather(b, b)
        for b in range(NB):
            process(b, b, first=True, last=False)

        @pl.loop(1, n_blocks - 1)
        def _(j):
            base = j * NB
            for b in range(NB):
                process(base + b, b, first=False, last=False)

        for b in range(NB):
            process((n_blocks - 1) * NB + b, b, first=False, last=True)
        for b in range(NB):
            wait_scatter(b)

    return sc_kernel


def kernel(x, table):
    n_rows, seq = x.shape
    table_p = jnp.pad(table, ((0, 0), (0, PADDED - D_MODEL)))
    out = _make_sc_gather(n_rows, seq)(x.reshape(-1), table_p)
    return out.reshape(n_rows, seq, D_MODEL)
